# Initial kernel scaffold; baseline (speedup 1.0000x reference)
import jax
import jax.numpy as jnp
from jax.experimental import pallas as pl


def kernel(p_u, adj_indices, adj_values, attn_indices, Wq, Wk, Wv, Wo, ln1_g, ln1_b, W1, b1, W2, b2, ln2_g, ln2_b):
    raise NotImplementedError("write your pallas kernel here")


# speedup vs baseline: 3.6930x; 3.6930x over previous
import functools

import jax
import jax.numpy as jnp
import numpy as np
from jax import lax
from jax.experimental import pallas as pl
from jax.experimental.pallas import tpu as pltpu
from jax.experimental.pallas import tpu_sc as plsc

U, D, T, E, H = 10000, 128, 32, 320000, 8
DH = D // H
FF = 4 * D

NC, NS = 2, 16
NW = NC * NS
CHUNK = 80

GCH = (U * T) // CHUNK
GPW = GCH // NW
ECH = E // CHUNK
EPW = ECH // NW
EG = 5
STRIPE = 632
UP = NS * STRIPE

@functools.cache
def _make_sc_gather():
    mesh = plsc.VectorSubcoreMesh(core_axis_name="c", subcore_axis_name="s")

    @functools.partial(
        pl.kernel,
        out_type=jax.ShapeDtypeStruct((U * T, D), jnp.float32),
        mesh=mesh,
        scratch_types=[
            pltpu.VMEM((GPW, CHUNK), jnp.int32),
            pltpu.VMEM((CHUNK, D), jnp.float32),
            pltpu.SemaphoreType.DMA,
        ],
    )
    def _sc_gather(table_hbm, idx_hbm, out_hbm, idx_v, buf, sem):
        wid = lax.axis_index("s") * NC + lax.axis_index("c")
        base = wid * GPW
        pltpu.sync_copy(idx_hbm.at[wid], idx_v)

        def body(j, _):
            pltpu.async_copy(table_hbm.at[idx_v.at[j]], buf, sem).wait()
            pltpu.sync_copy(buf, out_hbm.at[pl.ds((base + j) * CHUNK, CHUNK)])
            return 0

        lax.fori_loop(0, GPW, body, 0)

    return _sc_gather


BU = 80


def _ln_rows(x, g, b, eps=1e-5):
    mu = jnp.mean(x, axis=-1, keepdims=True)
    xc = x - mu
    var = jnp.mean(xc * xc, axis=-1, keepdims=True)
    return xc * jax.lax.rsqrt(var + eps) * g + b


def _tf_body(pu_ref, xs_ref, wq_ref, wk_ref, wv_ref, wo_ref, l1g_ref, l1b_ref,
             w1_ref, b1_ref, w2_ref, b2_ref, l2g_ref, l2b_ref, out_ref):
    x = pu_ref[...]
    xs = xs_ref[...]
    f32 = jnp.float32
    q = jnp.dot(x, wq_ref[...], preferred_element_type=f32)
    k = jnp.dot(xs, wk_ref[...], preferred_element_type=f32)
    v = jnp.dot(xs, wv_ref[...], preferred_element_type=f32)

    di = lax.broadcasted_iota(jnp.int32, (D, H), 0)
    hi = lax.broadcasted_iota(jnp.int32, (D, H), 1)
    seg = jnp.where(di // DH == hi, 1.0, 0.0).astype(f32)

    z = (q.reshape(BU, 1, D) * k.reshape(BU, T, D)).reshape(BU * T, D)
    s8 = jnp.dot(z, seg, preferred_element_type=f32) * (1.0 / np.sqrt(DH))
    s3 = s8.reshape(BU, T, H)
    m = jnp.max(s3, axis=1, keepdims=True)
    e = jnp.exp(s3 - m)
    a = e / jnp.sum(e, axis=1, keepdims=True)
    a_e = jnp.dot(a.reshape(BU * T, H), seg.T, preferred_element_type=f32)
    ctx = jnp.sum((a_e * v).reshape(BU, T, D), axis=1)

    o = jnp.dot(ctx, wo_ref[...], preferred_element_type=f32)
    x1 = _ln_rows(x + o, l1g_ref[...], l1b_ref[...])
    h1 = jnp.maximum(jnp.dot(x1, w1_ref[...], preferred_element_type=f32)
                     + b1_ref[...], 0.0)
    ff = jnp.dot(h1, w2_ref[...], preferred_element_type=f32) + b2_ref[...]
    out_ref[...] = _ln_rows(x1 + ff, l2g_ref[...], l2b_ref[...])


def _tc_transformer(p_u, xs, Wq, Wk, Wv, Wo, l1g, l1b, W1, b1, W2, b2, l2g, l2b):
    full = lambda shape: pl.BlockSpec(shape, lambda i: (0, 0))
    return pl.pallas_call(
        _tf_body,
        grid=(U // BU,),
        in_specs=[
            pl.BlockSpec((BU, D), lambda i: (i, 0)),
            pl.BlockSpec((BU * T, D), lambda i: (i, 0)),
            full((D, D)), full((D, D)), full((D, D)), full((D, D)),
            full((1, D)), full((1, D)),
            full((D, FF)), full((1, FF)),
            full((FF, D)), full((1, D)),
            full((1, D)), full((1, D)),
        ],
        out_specs=pl.BlockSpec((BU, D), lambda i: (i, 0)),
        out_shape=jax.ShapeDtypeStruct((U, D), jnp.float32),
    )(p_u, xs, Wq, Wk, Wv, Wo, l1g.reshape(1, D), l1b.reshape(1, D),
      W1, b1.reshape(1, FF), W2, b2.reshape(1, D),
      l2g.reshape(1, D), l2b.reshape(1, D))


@functools.cache
def _make_sc_gcn():
    mesh = plsc.VectorSubcoreMesh(core_axis_name="c", subcore_axis_name="s")

    @functools.partial(
        pl.kernel,
        out_type=jax.ShapeDtypeStruct((NC, UP, D), jnp.float32),
        mesh=mesh,
        scratch_types=[
            pltpu.VMEM((EG, CHUNK), jnp.int32),
            pltpu.VMEM((EG, CHUNK), jnp.int32),
            pltpu.VMEM((EG, CHUNK), jnp.float32),
            pltpu.VMEM((CHUNK, D), jnp.float32),
            pltpu.VMEM_SHARED((UP, D), jnp.float32),
            pltpu.SemaphoreType.DMA,
        ],
    )
    def _sc_gcn(x_hbm, cols_hbm, rows_hbm, vals_hbm, zeros_hbm, out_hbm,
                cols_v, rows_v, vals_v, buf, accum, sem):
        cid = lax.axis_index("c")
        sid = lax.axis_index("s")
        wid = sid * NC + cid
        pltpu.sync_copy(zeros_hbm.at[pl.ds(sid * STRIPE, STRIPE)],
                        accum.at[pl.ds(sid * STRIPE, STRIPE)])
        plsc.subcore_barrier()

        def chunk_body(j, _):
            pltpu.async_copy(x_hbm.at[cols_v.at[j]], buf, sem).wait()
            for g in range(CHUNK // 16):
                vv = vals_v[j, pl.ds(g * 16, 16)]
                for t in range(16):
                    val = vv[t]
                    e = g * 16 + t
                    for d8 in range(D // 16):
                        sl = pl.ds(d8 * 16, 16)
                        buf[e, sl] = buf[e, sl] * val
            pltpu.sync_copy(buf, accum.at[rows_v.at[j]], add=True)
            return 0

        def group_body(gi, _):
            pltpu.sync_copy(cols_hbm.at[wid, gi], cols_v)
            pltpu.sync_copy(rows_hbm.at[wid, gi], rows_v)
            pltpu.sync_copy(vals_hbm.at[wid, gi], vals_v)
            lax.fori_loop(0, EG, chunk_body, 0)
            return 0

        lax.fori_loop(0, EPW // EG, group_body, 0)
        plsc.subcore_barrier()
        pltpu.sync_copy(accum.at[pl.ds(sid * STRIPE, STRIPE)],
                        out_hbm.at[cid, pl.ds(sid * STRIPE, STRIPE)])

    return _sc_gcn


def _add_body(a_ref, b_ref, o_ref):
    o_ref[...] = a_ref[0] + b_ref[0]


def _tc_add(parts):
    blk = 2000
    return pl.pallas_call(
        _add_body,
        grid=(U // blk,),
        in_specs=[pl.BlockSpec((1, blk, D), lambda i: (0, i, 0)),
                  pl.BlockSpec((1, blk, D), lambda i: (1, i, 0))],
        out_specs=pl.BlockSpec((blk, D), lambda i: (i, 0)),
        out_shape=jax.ShapeDtypeStruct((U, D), jnp.float32),
    )(parts, parts)


def kernel(p_u, adj_indices, adj_values, attn_indices,
           Wq, Wk, Wv, Wo, ln1_g, ln1_b, W1, b1, W2, b2, ln2_g, ln2_b):
    ai = attn_indices.astype(jnp.int32).reshape(NW, GPW, CHUNK)
    xs = _make_sc_gather()(p_u, ai)
    p_tf = _tc_transformer(p_u, xs, Wq, Wk, Wv, Wo,
                           ln1_g, ln1_b, W1, b1, W2, b2, ln2_g, ln2_b)
    rows = adj_indices[0].astype(jnp.int32).reshape(NW, EPW // EG, EG, CHUNK)
    cols = adj_indices[1].astype(jnp.int32).reshape(NW, EPW // EG, EG, CHUNK)
    vals = adj_values.astype(jnp.float32).reshape(NW, EPW // EG, EG, CHUNK)
    parts = _make_sc_gcn()(p_tf, cols, rows, vals,
                           jnp.zeros((UP, D), jnp.float32))
    return _tc_add(parts)


# speedup vs baseline: 4.3311x; 1.1728x over previous
import functools

import jax
import jax.numpy as jnp
import numpy as np
from jax import lax
from jax.experimental import pallas as pl
from jax.experimental.pallas import tpu as pltpu
from jax.experimental.pallas import tpu_sc as plsc

U, D, T, E, H = 10000, 128, 32, 320000, 8
DH = D // H
FF = 4 * D

NC, NS = 2, 16
NW = NC * NS
CHUNK = 80

GCH = (U * T) // CHUNK
GPW = GCH // NW
ECH = E // CHUNK
EPW = ECH // NW
EG = 5
STRIPE = 632
UP = NS * STRIPE

@functools.cache
def _make_sc_gather():
    mesh = plsc.VectorSubcoreMesh(core_axis_name="c", subcore_axis_name="s")

    @functools.partial(
        pl.kernel,
        out_type=jax.ShapeDtypeStruct((U * T, D), jnp.float32),
        mesh=mesh,
        scratch_types=[
            pltpu.VMEM((GPW, CHUNK), jnp.int32),
            pltpu.VMEM((CHUNK, D), jnp.float32),
            pltpu.VMEM((CHUNK, D), jnp.float32),
            pltpu.SemaphoreType.DMA,
            pltpu.SemaphoreType.DMA,
        ],
    )
    def _sc_gather(table_hbm, idx_hbm, out_hbm, idx_v, buf0, buf1, sem0, sem1):
        wid = lax.axis_index("s") * NC + lax.axis_index("c")
        base = wid * GPW
        pltpu.sync_copy(idx_hbm.at[wid], idx_v)

        pltpu.async_copy(table_hbm.at[idx_v.at[0]], buf0, sem0)
        pltpu.async_copy(table_hbm.at[idx_v.at[1]], buf1, sem1)

        def pair_body(g, _):
            j0 = 2 * g
            pltpu.make_async_copy(table_hbm.at[idx_v.at[j0]], buf0, sem0).wait()
            pltpu.sync_copy(buf0, out_hbm.at[pl.ds((base + j0) * CHUNK, CHUNK)])
            pltpu.async_copy(table_hbm.at[idx_v.at[j0 + 2]], buf0, sem0)
            j1 = j0 + 1
            pltpu.make_async_copy(table_hbm.at[idx_v.at[j1]], buf1, sem1).wait()
            pltpu.sync_copy(buf1, out_hbm.at[pl.ds((base + j1) * CHUNK, CHUNK)])

            @pl.when(g < GPW // 2 - 1)
            def _():
                pltpu.async_copy(table_hbm.at[idx_v.at[j1 + 2]], buf1, sem1)

            return 0

        lax.fori_loop(0, GPW // 2, pair_body, 0)
        jt = GPW - 1
        pltpu.make_async_copy(table_hbm.at[idx_v.at[jt]], buf0, sem0).wait()
        pltpu.sync_copy(buf0, out_hbm.at[pl.ds((base + jt) * CHUNK, CHUNK)])

    return _sc_gather


BU = 80


def _ln_rows(x, g, b, eps=1e-5):
    mu = jnp.mean(x, axis=-1, keepdims=True)
    xc = x - mu
    var = jnp.mean(xc * xc, axis=-1, keepdims=True)
    return xc * jax.lax.rsqrt(var + eps) * g + b


def _tf_body(pu_ref, xs_ref, wq_ref, wk_ref, wv_ref, wo_ref, l1g_ref, l1b_ref,
             w1_ref, b1_ref, w2_ref, b2_ref, l2g_ref, l2b_ref, out_ref):
    x = pu_ref[...]
    xs = xs_ref[...]
    f32 = jnp.float32
    q = jnp.dot(x, wq_ref[...], preferred_element_type=f32)
    k = jnp.dot(xs, wk_ref[...], preferred_element_type=f32)
    v = jnp.dot(xs, wv_ref[...], preferred_element_type=f32)

    di = lax.broadcasted_iota(jnp.int32, (D, H), 0)
    hi = lax.broadcasted_iota(jnp.int32, (D, H), 1)
    seg = jnp.where(di // DH == hi, 1.0, 0.0).astype(f32)

    z = (q.reshape(BU, 1, D) * k.reshape(BU, T, D)).reshape(BU * T, D)
    s8 = jnp.dot(z, seg, preferred_element_type=f32) * (1.0 / np.sqrt(DH))
    s3 = s8.reshape(BU, T, H)
    m = jnp.max(s3, axis=1, keepdims=True)
    e = jnp.exp(s3 - m)
    a = e / jnp.sum(e, axis=1, keepdims=True)
    a_e = jnp.dot(a.reshape(BU * T, H), seg.T, preferred_element_type=f32)
    ctx = jnp.sum((a_e * v).reshape(BU, T, D), axis=1)

    o = jnp.dot(ctx, wo_ref[...], preferred_element_type=f32)
    x1 = _ln_rows(x + o, l1g_ref[...], l1b_ref[...])
    h1 = jnp.maximum(jnp.dot(x1, w1_ref[...], preferred_element_type=f32)
                     + b1_ref[...], 0.0)
    ff = jnp.dot(h1, w2_ref[...], preferred_element_type=f32) + b2_ref[...]
    out_ref[...] = _ln_rows(x1 + ff, l2g_ref[...], l2b_ref[...])


def _tc_transformer(p_u, xs, Wq, Wk, Wv, Wo, l1g, l1b, W1, b1, W2, b2, l2g, l2b):
    full = lambda shape: pl.BlockSpec(shape, lambda i: (0, 0))
    return pl.pallas_call(
        _tf_body,
        grid=(U // BU,),
        in_specs=[
            pl.BlockSpec((BU, D), lambda i: (i, 0)),
            pl.BlockSpec((BU * T, D), lambda i: (i, 0)),
            full((D, D)), full((D, D)), full((D, D)), full((D, D)),
            full((1, D)), full((1, D)),
            full((D, FF)), full((1, FF)),
            full((FF, D)), full((1, D)),
            full((1, D)), full((1, D)),
        ],
        out_specs=pl.BlockSpec((BU, D), lambda i: (i, 0)),
        out_shape=jax.ShapeDtypeStruct((U, D), jnp.float32),
    )(p_u, xs, Wq, Wk, Wv, Wo, l1g.reshape(1, D), l1b.reshape(1, D),
      W1, b1.reshape(1, FF), W2, b2.reshape(1, D),
      l2g.reshape(1, D), l2b.reshape(1, D))


EPP = 126
NPAIR = EPP // 2


@functools.cache
def _make_sc_gcn():
    mesh = plsc.VectorSubcoreMesh(core_axis_name="c", subcore_axis_name="s")

    @functools.partial(
        pl.kernel,
        out_type=jax.ShapeDtypeStruct((NC, UP, D), jnp.float32),
        mesh=mesh,
        scratch_types=[
            pltpu.VMEM((2, 2, 2, CHUNK), jnp.int32),
            pltpu.VMEM((2, 2, CHUNK), jnp.float32),
            pltpu.VMEM((CHUNK, D), jnp.float32),
            pltpu.VMEM((CHUNK, D), jnp.float32),
            pltpu.VMEM_SHARED((UP, D), jnp.float32),
            pltpu.SemaphoreType.DMA,
            pltpu.SemaphoreType.DMA,
            pltpu.SemaphoreType.DMA,
        ],
    )
    def _sc_gcn(x_hbm, idx_hbm, vals_hbm, zeros_hbm, out_hbm,
                ring, vring, buf0, buf1, accum, semg0, semg1, semi):
        cid = lax.axis_index("c")
        sid = lax.axis_index("s")
        wid = sid * NC + cid
        pltpu.sync_copy(zeros_hbm.at[pl.ds(sid * STRIPE, STRIPE)],
                        accum.at[pl.ds(sid * STRIPE, STRIPE)])
        plsc.subcore_barrier()

        pltpu.sync_copy(idx_hbm.at[wid, 0], ring.at[0])
        pltpu.sync_copy(vals_hbm.at[wid, 0], vring.at[0])
        pltpu.async_copy(idx_hbm.at[wid, 1], ring.at[1], semi)
        pltpu.async_copy(vals_hbm.at[wid, 1], vring.at[1], semi)
        pltpu.async_copy(x_hbm.at[ring.at[0, 0, 0]], buf0, semg0)
        pltpu.async_copy(x_hbm.at[ring.at[0, 0, 1]], buf1, semg1)

        def scale(buf, p, b):
            for g16 in range(CHUNK // 16):
                vv = vring[p, b, pl.ds(g16 * 16, 16)]
                for t in range(16):
                    val = vv[t]
                    e = g16 * 16 + t
                    for d8 in range(D // 16):
                        sl = pl.ds(d8 * 16, 16)
                        buf[e, sl] = buf[e, sl] * val

        def pair_body(g, _):
            p = lax.rem(g, 2)
            pn = 1 - p
            pltpu.make_async_copy(idx_hbm.at[wid, g + 1], ring.at[pn],
                                  semi).wait()
            pltpu.make_async_copy(vals_hbm.at[wid, g + 1], vring.at[pn],
                                  semi).wait()
            for b, buf, semg in ((0, buf0, semg0), (1, buf1, semg1)):
                pltpu.make_async_copy(x_hbm.at[ring.at[p, 0, b]], buf,
                                      semg).wait()
                scale(buf, p, b)
                pltpu.sync_copy(buf, accum.at[ring.at[p, 1, b]], add=True)

                @pl.when(g < NPAIR - 1)
                def _():
                    pltpu.async_copy(x_hbm.at[ring.at[pn, 0, b]], buf, semg)

            @pl.when(g < NPAIR - 1)
            def _():
                pltpu.async_copy(idx_hbm.at[wid, g + 2], ring.at[p], semi)
                pltpu.async_copy(vals_hbm.at[wid, g + 2], vring.at[p], semi)

            return 0

        lax.fori_loop(0, NPAIR, pair_body, 0)
        plsc.subcore_barrier()
        pltpu.sync_copy(accum.at[pl.ds(sid * STRIPE, STRIPE)],
                        out_hbm.at[cid, pl.ds(sid * STRIPE, STRIPE)])

    return _sc_gcn


def _add_body(a_ref, b_ref, o_ref):
    o_ref[...] = a_ref[0] + b_ref[0]


def _tc_add(parts):
    blk = 2000
    return pl.pallas_call(
        _add_body,
        grid=(U // blk,),
        in_specs=[pl.BlockSpec((1, blk, D), lambda i: (0, i, 0)),
                  pl.BlockSpec((1, blk, D), lambda i: (1, i, 0))],
        out_specs=pl.BlockSpec((blk, D), lambda i: (i, 0)),
        out_shape=jax.ShapeDtypeStruct((U, D), jnp.float32),
    )(parts, parts)


def kernel(p_u, adj_indices, adj_values, attn_indices,
           Wq, Wk, Wv, Wo, ln1_g, ln1_b, W1, b1, W2, b2, ln2_g, ln2_b):
    ai = attn_indices.astype(jnp.int32).reshape(NW, GPW, CHUNK)
    xs = _make_sc_gather()(p_u, ai)
    p_tf = _tc_transformer(p_u, xs, Wq, Wk, Wv, Wo,
                           ln1_g, ln1_b, W1, b1, W2, b2, ln2_g, ln2_b)
    rows = adj_indices[0].astype(jnp.int32).reshape(NW, EPW, CHUNK)
    cols = adj_indices[1].astype(jnp.int32).reshape(NW, EPW, CHUNK)
    vals = adj_values.astype(jnp.float32).reshape(NW, EPW, CHUNK)
    pad3 = lambda a: jnp.pad(a, ((0, 0), (0, EPP - EPW), (0, 0))
                             ).reshape(NW, NPAIR, 2, CHUNK)
    idx_all = jnp.stack([pad3(cols), pad3(rows)], axis=2)
    idx_all = jnp.pad(idx_all, ((0, 0), (0, 1), (0, 0), (0, 0), (0, 0)))
    vals_all = jnp.pad(pad3(vals), ((0, 0), (0, 1), (0, 0), (0, 0)))
    parts = _make_sc_gcn()(p_tf, idx_all, vals_all,
                           jnp.zeros((UP, D), jnp.float32))
    return _tc_add(parts)


# speedup vs baseline: 4.7958x; 1.1073x over previous
import functools

import jax
import jax.numpy as jnp
import numpy as np
from jax import lax
from jax.experimental import pallas as pl
from jax.experimental.pallas import tpu as pltpu
from jax.experimental.pallas import tpu_sc as plsc

U, D, T, E, H = 10000, 128, 32, 320000, 8
DH = D // H
FF = 4 * D

NC, NS = 2, 16
NW = NC * NS
CHUNK = 80

GCH = (U * T) // CHUNK
GPW = GCH // NW
ECH = E // CHUNK
EPW = ECH // NW
EG = 5
STRIPE = 632
UP = NS * STRIPE

@functools.cache
def _make_sc_gather():
    mesh = plsc.VectorSubcoreMesh(core_axis_name="c", subcore_axis_name="s")

    @functools.partial(
        pl.kernel,
        out_type=jax.ShapeDtypeStruct((U * T, D), jnp.float32),
        mesh=mesh,
        scratch_types=[
            pltpu.VMEM((GPW, CHUNK), jnp.int32),
            pltpu.VMEM((CHUNK, D), jnp.float32),
            pltpu.VMEM((CHUNK, D), jnp.float32),
            pltpu.SemaphoreType.DMA,
            pltpu.SemaphoreType.DMA,
            pltpu.SemaphoreType.DMA,
            pltpu.SemaphoreType.DMA,
        ],
    )
    def _sc_gather(table_hbm, idx_hbm, out_hbm, idx_v, buf0, buf1,
                   semg0, semg1, sems0, sems1):
        wid = lax.axis_index("s") * NC + lax.axis_index("c")
        base = wid * GPW
        pltpu.sync_copy(idx_hbm.at[wid], idx_v)

        pltpu.async_copy(table_hbm.at[idx_v.at[0]], buf0, semg0)
        pltpu.async_copy(table_hbm.at[idx_v.at[1]], buf1, semg1)

        def pair_body(g, _):
            j0 = 2 * g
            j1 = j0 + 1
            pltpu.make_async_copy(table_hbm.at[idx_v.at[j0]], buf0,
                                  semg0).wait()
            pltpu.async_copy(buf0, out_hbm.at[pl.ds((base + j0) * CHUNK,
                                                    CHUNK)], sems0)
            pltpu.make_async_copy(table_hbm.at[idx_v.at[j1]], buf1,
                                  semg1).wait()
            pltpu.async_copy(buf1, out_hbm.at[pl.ds((base + j1) * CHUNK,
                                                    CHUNK)], sems1)
            pltpu.make_async_copy(buf0, out_hbm.at[pl.ds((base + j0) * CHUNK,
                                                         CHUNK)], sems0).wait()
            pltpu.async_copy(table_hbm.at[idx_v.at[j0 + 2]], buf0, semg0)
            pltpu.make_async_copy(buf1, out_hbm.at[pl.ds((base + j1) * CHUNK,
                                                         CHUNK)], sems1).wait()

            @pl.when(g < GPW // 2 - 1)
            def _():
                pltpu.async_copy(table_hbm.at[idx_v.at[j1 + 2]], buf1, semg1)

            return 0

        lax.fori_loop(0, GPW // 2, pair_body, 0)
        jt = GPW - 1
        pltpu.make_async_copy(table_hbm.at[idx_v.at[jt]], buf0, semg0).wait()
        pltpu.sync_copy(buf0, out_hbm.at[pl.ds((base + jt) * CHUNK, CHUNK)])

    return _sc_gather


BU = 200


def _ln_rows(x, g, b, eps=1e-5):
    mu = jnp.mean(x, axis=-1, keepdims=True)
    xc = x - mu
    var = jnp.mean(xc * xc, axis=-1, keepdims=True)
    return xc * jax.lax.rsqrt(var + eps) * g + b


def _tf_body(pu_ref, xs_ref, wq_ref, wk_ref, wv_ref, wo_ref, l1g_ref, l1b_ref,
             w1_ref, b1_ref, w2_ref, b2_ref, l2g_ref, l2b_ref, out_ref):
    x = pu_ref[...]
    xs = xs_ref[...]
    f32 = jnp.float32
    q = jnp.dot(x, wq_ref[...], preferred_element_type=f32)
    k = jnp.dot(xs, wk_ref[...], preferred_element_type=f32)
    v = jnp.dot(xs, wv_ref[...], preferred_element_type=f32)

    di = lax.broadcasted_iota(jnp.int32, (D, H), 0)
    hi = lax.broadcasted_iota(jnp.int32, (D, H), 1)
    seg = jnp.where(di // DH == hi, 1.0, 0.0).astype(f32)

    z = (q.reshape(BU, 1, D) * k.reshape(BU, T, D)).reshape(BU * T, D)
    s8 = jnp.dot(z, seg, preferred_element_type=f32) * (1.0 / np.sqrt(DH))
    s3 = s8.reshape(BU, T, H)
    m = jnp.max(s3, axis=1, keepdims=True)
    e = jnp.exp(s3 - m)
    a = e / jnp.sum(e, axis=1, keepdims=True)
    a_e = jnp.dot(a.reshape(BU * T, H), seg.T, preferred_element_type=f32)
    ctx = jnp.sum((a_e * v).reshape(BU, T, D), axis=1)

    o = jnp.dot(ctx, wo_ref[...], preferred_element_type=f32)
    x1 = _ln_rows(x + o, l1g_ref[...], l1b_ref[...])
    h1 = jnp.maximum(jnp.dot(x1, w1_ref[...], preferred_element_type=f32)
                     + b1_ref[...], 0.0)
    ff = jnp.dot(h1, w2_ref[...], preferred_element_type=f32) + b2_ref[...]
    out_ref[...] = _ln_rows(x1 + ff, l2g_ref[...], l2b_ref[...])


def _tc_transformer(p_u, xs, Wq, Wk, Wv, Wo, l1g, l1b, W1, b1, W2, b2, l2g, l2b):
    full = lambda shape: pl.BlockSpec(shape, lambda i: (0, 0))
    return pl.pallas_call(
        _tf_body,
        grid=(U // BU,),
        in_specs=[
            pl.BlockSpec((BU, D), lambda i: (i, 0)),
            pl.BlockSpec((BU * T, D), lambda i: (i, 0)),
            full((D, D)), full((D, D)), full((D, D)), full((D, D)),
            full((1, D)), full((1, D)),
            full((D, FF)), full((1, FF)),
            full((FF, D)), full((1, D)),
            full((1, D)), full((1, D)),
        ],
        out_specs=pl.BlockSpec((BU, D), lambda i: (i, 0)),
        out_shape=jax.ShapeDtypeStruct((U, D), jnp.float32),
    )(p_u, xs, Wq, Wk, Wv, Wo, l1g.reshape(1, D), l1b.reshape(1, D),
      W1, b1.reshape(1, FF), W2, b2.reshape(1, D),
      l2g.reshape(1, D), l2b.reshape(1, D))


EPP = 126
NPAIR = EPP // 2


@functools.cache
def _make_sc_gcn():
    mesh = plsc.VectorSubcoreMesh(core_axis_name="c", subcore_axis_name="s")

    @functools.partial(
        pl.kernel,
        out_type=jax.ShapeDtypeStruct((NC, UP, D), jnp.float32),
        mesh=mesh,
        scratch_types=[
            pltpu.VMEM((2, 2, 2, CHUNK), jnp.int32),
            pltpu.VMEM((2, 2, CHUNK), jnp.float32),
            pltpu.VMEM((CHUNK, D), jnp.float32),
            pltpu.VMEM((CHUNK, D), jnp.float32),
            pltpu.VMEM_SHARED((UP, D), jnp.float32),
            pltpu.SemaphoreType.DMA,
            pltpu.SemaphoreType.DMA,
            pltpu.SemaphoreType.DMA,
            pltpu.SemaphoreType.DMA,
            pltpu.SemaphoreType.DMA,
        ],
    )
    def _sc_gcn(x_hbm, idx_hbm, vals_hbm, zeros_hbm, out_hbm,
                ring, vring, buf0, buf1, accum, semg0, semg1, semi,
                sems0, sems1):
        cid = lax.axis_index("c")
        sid = lax.axis_index("s")
        wid = sid * NC + cid
        pltpu.sync_copy(zeros_hbm.at[pl.ds(sid * STRIPE, STRIPE)],
                        accum.at[pl.ds(sid * STRIPE, STRIPE)])
        plsc.subcore_barrier()

        pltpu.sync_copy(idx_hbm.at[wid, 0], ring.at[0])
        pltpu.sync_copy(vals_hbm.at[wid, 0], vring.at[0])
        pltpu.async_copy(idx_hbm.at[wid, 1], ring.at[1], semi)
        pltpu.async_copy(vals_hbm.at[wid, 1], vring.at[1], semi)
        pltpu.async_copy(x_hbm.at[ring.at[0, 0, 0]], buf0, semg0)
        pltpu.async_copy(x_hbm.at[ring.at[0, 0, 1]], buf1, semg1)

        def scale(buf, p, b):
            for g16 in range(CHUNK // 16):
                vv = vring[p, b, pl.ds(g16 * 16, 16)]
                for t in range(16):
                    val = vv[t]
                    e = g16 * 16 + t
                    for d8 in range(D // 16):
                        sl = pl.ds(d8 * 16, 16)
                        buf[e, sl] = buf[e, sl] * val

        def pair_body(g, _):
            p = lax.rem(g, 2)
            pn = 1 - p
            pltpu.make_async_copy(idx_hbm.at[wid, g + 1], ring.at[pn],
                                  semi).wait()
            pltpu.make_async_copy(vals_hbm.at[wid, g + 1], vring.at[pn],
                                  semi).wait()
            for b, buf, semg, sems in ((0, buf0, semg0, sems0),
                                       (1, buf1, semg1, sems1)):
                pltpu.make_async_copy(x_hbm.at[ring.at[p, 0, b]], buf,
                                      semg).wait()
                scale(buf, p, b)
                pltpu.async_copy(buf, accum.at[ring.at[p, 1, b]], sems,
                                 add=True)
            for b, buf, semg, sems in ((0, buf0, semg0, sems0),
                                       (1, buf1, semg1, sems1)):
                pltpu.make_async_copy(buf, accum.at[ring.at[p, 1, b]],
                                      sems).wait()

                @pl.when(g < NPAIR - 1)
                def _():
                    pltpu.async_copy(x_hbm.at[ring.at[pn, 0, b]], buf, semg)

            @pl.when(g < NPAIR - 1)
            def _():
                pltpu.async_copy(idx_hbm.at[wid, g + 2], ring.at[p], semi)
                pltpu.async_copy(vals_hbm.at[wid, g + 2], vring.at[p], semi)

            return 0

        lax.fori_loop(0, NPAIR, pair_body, 0)
        plsc.subcore_barrier()
        pltpu.sync_copy(accum.at[pl.ds(sid * STRIPE, STRIPE)],
                        out_hbm.at[cid, pl.ds(sid * STRIPE, STRIPE)])

    return _sc_gcn


def _add_body(a_ref, b_ref, o_ref):
    o_ref[...] = a_ref[0] + b_ref[0]


def _tc_add(parts):
    blk = 2000
    return pl.pallas_call(
        _add_body,
        grid=(U // blk,),
        in_specs=[pl.BlockSpec((1, blk, D), lambda i: (0, i, 0)),
                  pl.BlockSpec((1, blk, D), lambda i: (1, i, 0))],
        out_specs=pl.BlockSpec((blk, D), lambda i: (i, 0)),
        out_shape=jax.ShapeDtypeStruct((U, D), jnp.float32),
    )(parts, parts)


def kernel(p_u, adj_indices, adj_values, attn_indices,
           Wq, Wk, Wv, Wo, ln1_g, ln1_b, W1, b1, W2, b2, ln2_g, ln2_b):
    ai = attn_indices.astype(jnp.int32).reshape(NW, GPW, CHUNK)
    xs = _make_sc_gather()(p_u, ai)
    p_tf = _tc_transformer(p_u, xs, Wq, Wk, Wv, Wo,
                           ln1_g, ln1_b, W1, b1, W2, b2, ln2_g, ln2_b)
    rows = adj_indices[0].astype(jnp.int32).reshape(NW, EPW, CHUNK)
    cols = adj_indices[1].astype(jnp.int32).reshape(NW, EPW, CHUNK)
    vals = adj_values.astype(jnp.float32).reshape(NW, EPW, CHUNK)
    pad3 = lambda a: jnp.pad(a, ((0, 0), (0, EPP - EPW), (0, 0))
                             ).reshape(NW, NPAIR, 2, CHUNK)
    idx_all = jnp.stack([pad3(cols), pad3(rows)], axis=2)
    idx_all = jnp.pad(idx_all, ((0, 0), (0, 1), (0, 0), (0, 0), (0, 0)))
    vals_all = jnp.pad(pad3(vals), ((0, 0), (0, 1), (0, 0), (0, 0)))
    parts = _make_sc_gcn()(p_tf, idx_all, vals_all,
                           jnp.zeros((UP, D), jnp.float32))
    return _tc_add(parts)


# speedup vs baseline: 5.0611x; 1.0553x over previous
import functools

import jax
import jax.numpy as jnp
import numpy as np
from jax import lax
from jax.experimental import pallas as pl
from jax.experimental.pallas import tpu as pltpu
from jax.experimental.pallas import tpu_sc as plsc

U, D, T, E, H = 10000, 128, 32, 320000, 8
DH = D // H
FF = 4 * D

NC, NS = 2, 16
NW = NC * NS
CHUNK = 80

GCH = (U * T) // CHUNK
GPW = GCH // NW
ECH = E // CHUNK
EPW = ECH // NW
EG = 5
STRIPE = 632
UP = NS * STRIPE

@functools.cache
def _make_sc_gather():
    mesh = plsc.VectorSubcoreMesh(core_axis_name="c", subcore_axis_name="s")

    @functools.partial(
        pl.kernel,
        out_type=jax.ShapeDtypeStruct((U * T, D), jnp.float32),
        mesh=mesh,
        scratch_types=[
            pltpu.VMEM((GPW, CHUNK), jnp.int32),
            pltpu.VMEM((CHUNK, D), jnp.float32),
            pltpu.VMEM((CHUNK, D), jnp.float32),
            pltpu.SemaphoreType.DMA,
            pltpu.SemaphoreType.DMA,
        ],
    )
    def _sc_gather(table_hbm, idx_hbm, out_hbm, idx_v, buf0, buf1,
                   semg0, semg1):
        wid = lax.axis_index("s") * NC + lax.axis_index("c")
        base = wid * GPW
        pltpu.sync_copy(idx_hbm.at[wid], idx_v)

        pltpu.async_copy(table_hbm.at[idx_v.at[0]], buf0, semg0)
        pltpu.async_copy(table_hbm.at[idx_v.at[1]], buf1, semg1)

        def pair_body(g, _):
            j0 = 2 * g
            j1 = j0 + 1
            pltpu.make_async_copy(table_hbm.at[idx_v.at[j0]], buf0,
                                  semg0).wait()
            pltpu.sync_copy(buf0, out_hbm.at[pl.ds((base + j0) * CHUNK,
                                                   CHUNK)])
            pltpu.async_copy(table_hbm.at[idx_v.at[j0 + 2]], buf0, semg0)
            pltpu.make_async_copy(table_hbm.at[idx_v.at[j1]], buf1,
                                  semg1).wait()
            pltpu.sync_copy(buf1, out_hbm.at[pl.ds((base + j1) * CHUNK,
                                                   CHUNK)])

            @pl.when(g < GPW // 2 - 1)
            def _():
                pltpu.async_copy(table_hbm.at[idx_v.at[j1 + 2]], buf1, semg1)

            return 0

        lax.fori_loop(0, GPW // 2, pair_body, 0)
        jt = GPW - 1
        pltpu.make_async_copy(table_hbm.at[idx_v.at[jt]], buf0, semg0).wait()
        pltpu.sync_copy(buf0, out_hbm.at[pl.ds((base + jt) * CHUNK, CHUNK)])

    return _sc_gather


BU = 400


def _ln_rows(x, g, b, eps=1e-5):
    mu = jnp.mean(x, axis=-1, keepdims=True)
    xc = x - mu
    var = jnp.mean(xc * xc, axis=-1, keepdims=True)
    return xc * jax.lax.rsqrt(var + eps) * g + b


def _tf_body(pu_ref, xs_ref, wq_ref, wk_ref, wv_ref, wo_ref, l1g_ref, l1b_ref,
             w1_ref, b1_ref, w2_ref, b2_ref, l2g_ref, l2b_ref, out_ref):
    x = pu_ref[...]
    xs = xs_ref[...]
    f32 = jnp.float32
    q = jnp.dot(x, wq_ref[...], preferred_element_type=f32)
    k = jnp.dot(xs, wk_ref[...], preferred_element_type=f32)
    v = jnp.dot(xs, wv_ref[...], preferred_element_type=f32)

    di = lax.broadcasted_iota(jnp.int32, (D, H), 0)
    hi = lax.broadcasted_iota(jnp.int32, (D, H), 1)
    seg = jnp.where(di // DH == hi, 1.0, 0.0).astype(f32)

    z = (q.reshape(BU, 1, D) * k.reshape(BU, T, D)).reshape(BU * T, D)
    s8 = jnp.dot(z, seg, preferred_element_type=f32) * (1.0 / np.sqrt(DH))
    s3 = s8.reshape(BU, T, H)
    m = jnp.max(s3, axis=1, keepdims=True)
    e = jnp.exp(s3 - m)
    a = e / jnp.sum(e, axis=1, keepdims=True)
    a_e = jnp.dot(a.reshape(BU * T, H), seg.T, preferred_element_type=f32)
    ctx = jnp.sum((a_e * v).reshape(BU, T, D), axis=1)

    o = jnp.dot(ctx, wo_ref[...], preferred_element_type=f32)
    x1 = _ln_rows(x + o, l1g_ref[...], l1b_ref[...])
    h1 = jnp.maximum(jnp.dot(x1, w1_ref[...], preferred_element_type=f32)
                     + b1_ref[...], 0.0)
    ff = jnp.dot(h1, w2_ref[...], preferred_element_type=f32) + b2_ref[...]
    out_ref[...] = _ln_rows(x1 + ff, l2g_ref[...], l2b_ref[...])


def _tc_transformer(p_u, xs, Wq, Wk, Wv, Wo, l1g, l1b, W1, b1, W2, b2, l2g, l2b):
    full = lambda shape: pl.BlockSpec(shape, lambda i: (0, 0))
    return pl.pallas_call(
        _tf_body,
        grid=(U // BU,),
        in_specs=[
            pl.BlockSpec((BU, D), lambda i: (i, 0)),
            pl.BlockSpec((BU * T, D), lambda i: (i, 0)),
            full((D, D)), full((D, D)), full((D, D)), full((D, D)),
            full((1, D)), full((1, D)),
            full((D, FF)), full((1, FF)),
            full((FF, D)), full((1, D)),
            full((1, D)), full((1, D)),
        ],
        out_specs=pl.BlockSpec((BU, D), lambda i: (i, 0)),
        out_shape=jax.ShapeDtypeStruct((U, D), jnp.float32),
    )(p_u, xs, Wq, Wk, Wv, Wo, l1g.reshape(1, D), l1b.reshape(1, D),
      W1, b1.reshape(1, FF), W2, b2.reshape(1, D),
      l2g.reshape(1, D), l2b.reshape(1, D))


EPP = 126
NPAIR = EPP // 2


@functools.cache
def _make_sc_gcn():
    mesh = plsc.VectorSubcoreMesh(core_axis_name="c", subcore_axis_name="s")

    @functools.partial(
        pl.kernel,
        out_type=jax.ShapeDtypeStruct((NC, UP, D), jnp.float32),
        mesh=mesh,
        scratch_types=[
            pltpu.VMEM((2, 2, 2, CHUNK), jnp.int32),
            pltpu.VMEM((2, 2, CHUNK), jnp.float32),
            pltpu.VMEM((CHUNK, D), jnp.float32),
            pltpu.VMEM((CHUNK, D), jnp.float32),
            pltpu.VMEM_SHARED((UP, D), jnp.float32),
            pltpu.SemaphoreType.DMA,
            pltpu.SemaphoreType.DMA,
            pltpu.SemaphoreType.DMA,
            pltpu.SemaphoreType.DMA,
            pltpu.SemaphoreType.DMA,
        ],
    )
    def _sc_gcn(x_hbm, idx_hbm, vals_hbm, zeros_hbm, out_hbm,
                ring, vring, buf0, buf1, accum, semg0, semg1, semi,
                sems0, sems1):
        cid = lax.axis_index("c")
        sid = lax.axis_index("s")
        wid = sid * NC + cid
        pltpu.sync_copy(zeros_hbm.at[pl.ds(sid * STRIPE, STRIPE)],
                        accum.at[pl.ds(sid * STRIPE, STRIPE)])
        plsc.subcore_barrier()

        pltpu.sync_copy(idx_hbm.at[wid, 0], ring.at[0])
        pltpu.sync_copy(vals_hbm.at[wid, 0], vring.at[0])
        pltpu.async_copy(idx_hbm.at[wid, 1], ring.at[1], semi)
        pltpu.async_copy(vals_hbm.at[wid, 1], vring.at[1], semi)
        pltpu.async_copy(x_hbm.at[ring.at[0, 0, 0]], buf0, semg0)
        pltpu.async_copy(x_hbm.at[ring.at[0, 0, 1]], buf1, semg1)

        def scale(buf, p, b):
            for g16 in range(CHUNK // 16):
                vv = vring[p, b, pl.ds(g16 * 16, 16)]
                for t in range(16):
                    bc = vv.at[jnp.full((16,), t, jnp.int32)].get(
                        mode="promise_in_bounds")
                    e = g16 * 16 + t
                    for d8 in range(D // 16):
                        sl = pl.ds(d8 * 16, 16)
                        buf[e, sl] = buf[e, sl] * bc

        def pair_body(g, _):
            p = lax.rem(g, 2)
            pn = 1 - p
            pltpu.make_async_copy(idx_hbm.at[wid, g + 1], ring.at[pn],
                                  semi).wait()
            pltpu.make_async_copy(vals_hbm.at[wid, g + 1], vring.at[pn],
                                  semi).wait()
            for b, buf, semg, sems in ((0, buf0, semg0, sems0),
                                       (1, buf1, semg1, sems1)):
                pltpu.make_async_copy(x_hbm.at[ring.at[p, 0, b]], buf,
                                      semg).wait()
                scale(buf, p, b)
                pltpu.async_copy(buf, accum.at[ring.at[p, 1, b]], sems,
                                 add=True)
            for b, buf, semg, sems in ((0, buf0, semg0, sems0),
                                       (1, buf1, semg1, sems1)):
                pltpu.make_async_copy(buf, accum.at[ring.at[p, 1, b]],
                                      sems).wait()

                @pl.when(g < NPAIR - 1)
                def _():
                    pltpu.async_copy(x_hbm.at[ring.at[pn, 0, b]], buf, semg)

            @pl.when(g < NPAIR - 1)
            def _():
                pltpu.async_copy(idx_hbm.at[wid, g + 2], ring.at[p], semi)
                pltpu.async_copy(vals_hbm.at[wid, g + 2], vring.at[p], semi)

            return 0

        lax.fori_loop(0, NPAIR, pair_body, 0)
        plsc.subcore_barrier()
        pltpu.sync_copy(accum.at[pl.ds(sid * STRIPE, STRIPE)],
                        out_hbm.at[cid, pl.ds(sid * STRIPE, STRIPE)])

    return _sc_gcn


def _add_body(a_ref, b_ref, o_ref):
    o_ref[...] = a_ref[0] + b_ref[0]


def _tc_add(parts):
    blk = 2000
    return pl.pallas_call(
        _add_body,
        grid=(U // blk,),
        in_specs=[pl.BlockSpec((1, blk, D), lambda i: (0, i, 0)),
                  pl.BlockSpec((1, blk, D), lambda i: (1, i, 0))],
        out_specs=pl.BlockSpec((blk, D), lambda i: (i, 0)),
        out_shape=jax.ShapeDtypeStruct((U, D), jnp.float32),
    )(parts, parts)


def kernel(p_u, adj_indices, adj_values, attn_indices,
           Wq, Wk, Wv, Wo, ln1_g, ln1_b, W1, b1, W2, b2, ln2_g, ln2_b):
    ai = attn_indices.astype(jnp.int32).reshape(NW, GPW, CHUNK)
    xs = _make_sc_gather()(p_u, ai)
    p_tf = _tc_transformer(p_u, xs, Wq, Wk, Wv, Wo,
                           ln1_g, ln1_b, W1, b1, W2, b2, ln2_g, ln2_b)
    rows = adj_indices[0].astype(jnp.int32).reshape(NW, EPW, CHUNK)
    cols = adj_indices[1].astype(jnp.int32).reshape(NW, EPW, CHUNK)
    vals = adj_values.astype(jnp.float32).reshape(NW, EPW, CHUNK)
    pad3 = lambda a: jnp.pad(a, ((0, 0), (0, EPP - EPW), (0, 0))
                             ).reshape(NW, NPAIR, 2, CHUNK)
    idx_all = jnp.stack([pad3(cols), pad3(rows)], axis=2)
    idx_all = jnp.pad(idx_all, ((0, 0), (0, 1), (0, 0), (0, 0), (0, 0)))
    vals_all = jnp.pad(pad3(vals), ((0, 0), (0, 1), (0, 0), (0, 0)))
    parts = _make_sc_gcn()(p_tf, idx_all, vals_all,
                           jnp.zeros((UP, D), jnp.float32))
    return _tc_add(parts)
